# baseline (speedup 1.0000x reference)
import jax
import jax.numpy as jnp
from jax import lax
from jax.experimental import pallas as pl
from jax.experimental.pallas import tpu as pltpu
from jax.experimental.pallas import tpu_sc as plsc

_NC, _NS = 2, 16
_NW = _NC * _NS
_L = 16


def kernel(input_tensor, indices):
    B, N, D = input_tensor.shape
    S = indices.shape[1]
    assert indices.shape[0] == B

    chunk = 128
    idx_rows = (B * S) // chunk
    rpw = idx_rows // _NW
    rows_per_batch = S // chunk
    assert S % chunk == 0 and idx_rows % _NW == 0 and rpw % rows_per_batch == 0
    assert chunk % _L == 0 and D % _L == 0

    tbl = input_tensor.reshape(B * N, D)
    idxr = indices.astype(jnp.int32).reshape(idx_rows, chunk)

    npair = 3

    def body(tbl, idxr, out, idx_v, *rest):
        bufs = rest[:npair]
        gsems = rest[npair:npair + 6]
        osems = rest[npair + 6:]
        wid = lax.axis_index("s") * _NC + lax.axis_index("c")
        r0 = wid * rpw
        head = min(8, rpw)
        pltpu.sync_copy(idxr.at[pl.ds(r0, head)], idx_v.at[pl.ds(0, head)])

        def rebase(j):
            base = (wid * (rpw // rows_per_batch) + j // rows_per_batch) * N
            for k in range(chunk // _L):
                sl = pl.ds(k * _L, _L)
                idx_v[j, sl] = idx_v[j, sl] + base

        def gather(j):
            cp = pltpu.make_async_copy(
                tbl.at[idx_v.at[j]],
                bufs[(j // 2) % npair].at[pl.ds((j % 2) * chunk, chunk)],
                gsems[j % 6])
            cp.start()
            return cp

        def out_copy(p):
            cp = pltpu.make_async_copy(
                bufs[p % npair],
                out.at[pl.ds((r0 + 2 * p) * chunk, 2 * chunk)],
                osems[p % npair])
            cp.start()
            return cp

        npairs = rpw // 2
        dp = 2
        assert 2 * dp <= head
        gcps = []
        for j in range(2 * dp):
            rebase(j)
            gcps.append(gather(j))
        if rpw > head:
            pltpu.sync_copy(idxr.at[pl.ds(r0 + head, rpw - head)],
                            idx_v.at[pl.ds(head, rpw - head)])
        ocps = []
        owaited = [False] * npairs
        for p in range(npairs):
            if p + dp < npairs:
                rebase(2 * (p + dp))
                rebase(2 * (p + dp) + 1)
            gcps[2 * p].wait()
            gcps[2 * p + 1].wait()
            ocps.append(out_copy(p))
            if p + dp < npairs:
                if p + dp - npair >= 0:
                    ocps[p + dp - npair].wait()
                    owaited[p + dp - npair] = True
                gcps.append(gather(2 * (p + dp)))
                gcps.append(gather(2 * (p + dp) + 1))
        for p in range(npairs):
            if not owaited[p]:
                ocps[p].wait()

    mesh = plsc.VectorSubcoreMesh(
        core_axis_name="c", subcore_axis_name="s",
        num_cores=_NC, num_subcores=_NS)
    out = pl.kernel(
        body,
        out_type=jax.ShapeDtypeStruct((B * S, D), jnp.float32),
        mesh=mesh,
        scratch_types=(
            [pltpu.VMEM((rpw, chunk), jnp.int32)]
            + [pltpu.VMEM((2 * chunk, D), jnp.float32)] * npair
            + [pltpu.SemaphoreType.DMA] * (6 + npair)
        ),
    )(tbl, idxr)
    return out.reshape(B, S, D)


# speedup vs baseline: 1.0357x; 1.0357x over previous
import jax
import jax.numpy as jnp
from jax import lax
from jax.experimental import pallas as pl
from jax.experimental.pallas import tpu as pltpu
from jax.experimental.pallas import tpu_sc as plsc

_NC, _NS = 2, 16
_NW = _NC * _NS
_L = 16


def kernel(input_tensor, indices):
    B, N, D = input_tensor.shape
    S = indices.shape[1]
    assert indices.shape[0] == B

    chunk = 128
    idx_rows = (B * S) // chunk
    rpw = idx_rows // _NW
    rows_per_batch = S // chunk
    assert S % chunk == 0 and idx_rows % _NW == 0 and rpw % rows_per_batch == 0
    assert chunk % _L == 0 and D % _L == 0

    tbl = input_tensor.reshape(B * N, D)
    idxr = indices.astype(jnp.int32).reshape(idx_rows, chunk)

    nbuf = 7

    def body(tbl, idxr, out, idx_v, *rest):
        bufs, gsems, osems = rest[:nbuf], rest[nbuf:2 * nbuf], rest[2 * nbuf:]
        wid = lax.axis_index("s") * _NC + lax.axis_index("c")
        r0 = wid * rpw
        head = min(8, rpw)
        pltpu.sync_copy(idxr.at[pl.ds(r0, head)], idx_v.at[pl.ds(0, head)])

        def rebase(j):
            base = (wid * (rpw // rows_per_batch) + j // rows_per_batch) * N
            for k in range(chunk // _L):
                sl = pl.ds(k * _L, _L)
                idx_v[j, sl] = idx_v[j, sl] + base

        def gather(j):
            b = j % nbuf
            cp = pltpu.make_async_copy(tbl.at[idx_v.at[j]], bufs[b], gsems[b])
            cp.start()
            return cp

        def out_copy(j):
            b = j % nbuf
            cp = pltpu.make_async_copy(
                bufs[b], out.at[pl.ds((r0 + j) * chunk, chunk)], osems[b])
            cp.start()
            return cp

        depth = 5
        assert depth <= head
        gcps = []
        for j in range(min(depth, rpw)):
            rebase(j)
            gcps.append(gather(j))
        if rpw > head:
            pltpu.sync_copy(idxr.at[pl.ds(r0 + head, rpw - head)],
                            idx_v.at[pl.ds(head, rpw - head)])
        ocps = []
        owaited = [False] * rpw
        for j in range(rpw):
            if j + depth < rpw:
                rebase(j + depth)
            gcps[j].wait()
            ocps.append(out_copy(j))
            if j + depth < rpw:
                if j + depth - nbuf >= 0:
                    ocps[j + depth - nbuf].wait()
                    owaited[j + depth - nbuf] = True
                gcps.append(gather(j + depth))
        for j in range(rpw):
            if not owaited[j]:
                ocps[j].wait()

    mesh = plsc.VectorSubcoreMesh(
        core_axis_name="c", subcore_axis_name="s",
        num_cores=_NC, num_subcores=_NS)
    out = pl.kernel(
        body,
        out_type=jax.ShapeDtypeStruct((B * S, D), jnp.float32),
        mesh=mesh,
        scratch_types=(
            [pltpu.VMEM((rpw, chunk), jnp.int32)]
            + [pltpu.VMEM((chunk, D), jnp.float32)] * nbuf
            + [pltpu.SemaphoreType.DMA] * (2 * nbuf)
        ),
    )(tbl, idxr)
    return out.reshape(B, S, D)
